# baseline (speedup 1.0000x reference)
import functools

import jax
import jax.numpy as jnp
from jax import lax
from jax.experimental import pallas as pl
from jax.experimental.pallas import tpu as pltpu
from jax.experimental.pallas import tpu_sc as plsc

EMB = 64
HID = 128
EPS = 1e-12

NC = 2
NS = 16
NW = NC * NS

CHUNK = 1024


def _gather_body(table_hbm, idx_hbm, out_hbm, idx_v, rows_v, sem, *, b_per_w,
                 n_chunks):
    wid = lax.axis_index("s") * NC + lax.axis_index("c")
    base = wid * b_per_w

    def body(i, carry):
        off = base + i * CHUNK
        pltpu.sync_copy(idx_hbm.at[pl.ds(off, CHUNK)], idx_v)
        pltpu.async_copy(table_hbm.at[idx_v], rows_v, sem).wait()
        pltpu.sync_copy(rows_v, out_hbm.at[pl.ds(off, CHUNK)])
        return carry

    lax.fori_loop(0, n_chunks, body, 0)


def _sc_gather(table, idx_flat):
    (b,) = idx_flat.shape
    assert b % (NW * CHUNK) == 0, b
    b_per_w = b // NW
    n_chunks = b_per_w // CHUNK
    mesh = plsc.VectorSubcoreMesh(core_axis_name="c", subcore_axis_name="s",
                                  num_cores=NC, num_subcores=NS)
    f = pl.kernel(
        functools.partial(_gather_body, b_per_w=b_per_w, n_chunks=n_chunks),
        out_type=jax.ShapeDtypeStruct((b, EMB), jnp.float32),
        mesh=mesh,
        scratch_types=[
            pltpu.VMEM((CHUNK,), jnp.int32),
            pltpu.VMEM((CHUNK, EMB), jnp.float32),
            pltpu.SemaphoreType.DMA,
        ],
        compiler_params=pltpu.CompilerParams(use_tc_tiling_on_sc=False),
    )
    return f(table, idx_flat)


def _mlp_ln_body(x_ref, w_ref, b_ref, g_ref, beta_ref, o_ref):
    x = x_ref[...]
    h = jnp.dot(x, w_ref[...], preferred_element_type=jnp.float32) + b_ref[...]
    mean = jnp.mean(h, axis=-1, keepdims=True)
    xc = h - mean
    var = jnp.mean(xc * xc, axis=-1, keepdims=True)
    o_ref[...] = xc * lax.rsqrt(var + EPS) * g_ref[...] + beta_ref[...]


def _tc_mlp_ln(x, W, b, gamma, beta, blk):
    n = x.shape[0]
    assert n % blk == 0
    grid = (n // blk,)
    return pl.pallas_call(
        _mlp_ln_body,
        grid=grid,
        in_specs=[
            pl.BlockSpec((blk, EMB), lambda i: (i, 0)),
            pl.BlockSpec((EMB, HID), lambda i: (0, 0)),
            pl.BlockSpec((1, HID), lambda i: (0, 0)),
            pl.BlockSpec((1, HID), lambda i: (0, 0)),
            pl.BlockSpec((1, HID), lambda i: (0, 0)),
        ],
        out_specs=pl.BlockSpec((blk, HID), lambda i: (i, 0)),
        out_shape=jax.ShapeDtypeStruct((n, HID), jnp.float32),
        compiler_params=pltpu.CompilerParams(
            dimension_semantics=("arbitrary",)),
    )(x, W, b.reshape(1, HID), gamma.reshape(1, HID), beta.reshape(1, HID))


def kernel(input_ids, table, W, b, gamma, beta):
    B, L = input_ids.shape
    ids_flat = input_ids.reshape(-1).astype(jnp.int32)
    gathered = _sc_gather(table, ids_flat)
    out = _tc_mlp_ln(gathered, W, b, gamma, beta, blk=4096)
    return out.reshape(B, L, HID)


# speedup vs baseline: 1.0655x; 1.0655x over previous
import functools

import jax
import jax.numpy as jnp
from jax import lax
from jax.experimental import pallas as pl
from jax.experimental.pallas import tpu as pltpu
from jax.experimental.pallas import tpu_sc as plsc

EMB = 64
HID = 128
EPS = 1e-12

NC = 2
NS = 16
NW = NC * NS

CHUNK = 1024


def _gather_body(table_hbm, idx_hbm, out_hbm, idx_v, rows_v, sem, *, b_per_w,
                 n_chunks):
    wid = lax.axis_index("s") * NC + lax.axis_index("c")
    base = wid * b_per_w

    def body(i, carry):
        off = base + i * CHUNK
        pltpu.sync_copy(idx_hbm.at[pl.ds(off, CHUNK)], idx_v)
        pltpu.async_copy(table_hbm.at[idx_v], rows_v, sem).wait()
        pltpu.sync_copy(rows_v, out_hbm.at[pl.ds(off, CHUNK)])
        return carry

    lax.fori_loop(0, n_chunks, body, 0)


def _sc_gather_pairs(table, idx_flat):
    (b,) = idx_flat.shape
    assert b % (NW * CHUNK) == 0, b
    b_per_w = b // NW
    n_chunks = b_per_w // CHUNK
    mesh = plsc.VectorSubcoreMesh(core_axis_name="c", subcore_axis_name="s",
                                  num_cores=NC, num_subcores=NS)
    f = pl.kernel(
        functools.partial(_gather_body, b_per_w=b_per_w, n_chunks=n_chunks),
        out_type=jax.ShapeDtypeStruct((b, EMB), jnp.float32),
        mesh=mesh,
        scratch_types=[
            pltpu.VMEM((CHUNK,), jnp.int32),
            pltpu.VMEM((CHUNK, EMB), jnp.float32),
            pltpu.SemaphoreType.DMA,
        ],
        compiler_params=pltpu.CompilerParams(use_tc_tiling_on_sc=False),
    )
    return f(table, idx_flat)


def _mlp_ln_pairs_body(x_ref, wa_ref, wb_ref, b_ref, g_ref, beta_ref, o_ref):
    x2 = x_ref[...]
    b = b_ref[...]
    he = jnp.dot(x2, wa_ref[...], preferred_element_type=jnp.float32) + b
    ho = jnp.dot(x2, wb_ref[...], preferred_element_type=jnp.float32) + b

    def ln(h):
        mean = jnp.mean(h, axis=-1, keepdims=True)
        xc = h - mean
        var = jnp.mean(xc * xc, axis=-1, keepdims=True)
        return xc * lax.rsqrt(var + EPS) * g_ref[...] + beta_ref[...]

    ye = ln(he)
    yo = ln(ho)
    n2 = ye.shape[0]
    y = jnp.concatenate([ye[:, None, :], yo[:, None, :]], axis=1)
    o_ref[...] = y.reshape(2 * n2, HID)


def _tc_mlp_ln_pairs(x2, W, b, gamma, beta, blk2):
    n2 = x2.shape[0]
    assert n2 % blk2 == 0
    zeros = jnp.zeros((EMB, HID), jnp.float32)
    wa = jnp.concatenate([W, zeros], axis=0)
    wb = jnp.concatenate([zeros, W], axis=0)
    grid = (n2 // blk2,)
    return pl.pallas_call(
        _mlp_ln_pairs_body,
        grid=grid,
        in_specs=[
            pl.BlockSpec((blk2, 2 * EMB), lambda i: (i, 0)),
            pl.BlockSpec((2 * EMB, HID), lambda i: (0, 0)),
            pl.BlockSpec((2 * EMB, HID), lambda i: (0, 0)),
            pl.BlockSpec((1, HID), lambda i: (0, 0)),
            pl.BlockSpec((1, HID), lambda i: (0, 0)),
            pl.BlockSpec((1, HID), lambda i: (0, 0)),
        ],
        out_specs=pl.BlockSpec((2 * blk2, HID), lambda i: (i, 0)),
        out_shape=jax.ShapeDtypeStruct((2 * n2, HID), jnp.float32),
        compiler_params=pltpu.CompilerParams(
            dimension_semantics=("arbitrary",)),
    )(x2, wa, wb, b.reshape(1, HID), gamma.reshape(1, HID),
      beta.reshape(1, HID))


def kernel(input_ids, table, W, b, gamma, beta):
    B, L = input_ids.shape
    ids_flat = input_ids.reshape(-1).astype(jnp.int32)
    gathered2 = _sc_gather_pairs(table, ids_flat).reshape(-1, 2 * EMB)
    out = _tc_mlp_ln_pairs(gathered2, W, b, gamma, beta, blk2=2048)
    return out.reshape(B, L, HID)


# speedup vs baseline: 1.4564x; 1.3669x over previous
import functools

import jax
import jax.numpy as jnp
from jax import lax
from jax.experimental import pallas as pl
from jax.experimental.pallas import tpu as pltpu
from jax.experimental.pallas import tpu_sc as plsc

EMB = 64
HID = 128
EPS = 1e-12

NC = 2
NS = 16
NW = NC * NS

CHUNK = 800

TBLK = 8000


def _dense_body(t_ref, w_ref, b_ref, g_ref, beta_ref, o_ref):
    x = t_ref[...]
    hc = jnp.dot(x, w_ref[...], preferred_element_type=jnp.float32)
    hc = hc + b_ref[...]
    var = jnp.mean(hc * hc, axis=-1, keepdims=True)
    o_ref[...] = hc * lax.rsqrt(var + EPS) * g_ref[...] + beta_ref[...]


def _tc_transform_table(table, W, b, gamma, beta):
    v = table.shape[0]
    assert v % TBLK == 0
    wc = W - jnp.mean(W, axis=1, keepdims=True)
    bc = (b - jnp.mean(b)).reshape(1, HID)
    grid = (v // TBLK,)
    return pl.pallas_call(
        _dense_body,
        grid=grid,
        in_specs=[
            pl.BlockSpec((TBLK, EMB), lambda i: (i, 0)),
            pl.BlockSpec((EMB, HID), lambda i: (0, 0)),
            pl.BlockSpec((1, HID), lambda i: (0, 0)),
            pl.BlockSpec((1, HID), lambda i: (0, 0)),
            pl.BlockSpec((1, HID), lambda i: (0, 0)),
        ],
        out_specs=pl.BlockSpec((TBLK, HID), lambda i: (i, 0)),
        out_shape=jax.ShapeDtypeStruct((v, HID), jnp.float32),
        compiler_params=pltpu.CompilerParams(
            dimension_semantics=("arbitrary",)),
    )(table, wc, bc, gamma.reshape(1, HID), beta.reshape(1, HID))


def _gather_body(f_hbm, idx_hbm, out_hbm, idx_v, rows_v, sem, *, b_per_w,
                 n_chunks):
    wid = lax.axis_index("s") * NC + lax.axis_index("c")
    base = wid * b_per_w

    def body(i, carry):
        off = base + i * CHUNK
        pltpu.sync_copy(idx_hbm.at[pl.ds(off, CHUNK)], idx_v)
        pltpu.async_copy(f_hbm.at[idx_v], rows_v, sem).wait()
        pltpu.sync_copy(rows_v, out_hbm.at[pl.ds(off, CHUNK)])
        return carry

    lax.fori_loop(0, n_chunks, body, 0)


def _sc_gather_rows(f, idx_flat):
    (b,) = idx_flat.shape
    assert b % (NW * CHUNK) == 0, b
    b_per_w = b // NW
    n_chunks = b_per_w // CHUNK
    mesh = plsc.VectorSubcoreMesh(core_axis_name="c", subcore_axis_name="s",
                                  num_cores=NC, num_subcores=NS)
    f_call = pl.kernel(
        functools.partial(_gather_body, b_per_w=b_per_w, n_chunks=n_chunks),
        out_type=jax.ShapeDtypeStruct((b, HID), jnp.float32),
        mesh=mesh,
        scratch_types=[
            pltpu.VMEM((CHUNK,), jnp.int32),
            pltpu.VMEM((CHUNK, HID), jnp.float32),
            pltpu.SemaphoreType.DMA,
        ],
    )
    return f_call(f, idx_flat)


def kernel(input_ids, table, W, b, gamma, beta):
    B, L = input_ids.shape
    ids_flat = input_ids.reshape(-1).astype(jnp.int32)
    f = _tc_transform_table(table, W, b, gamma, beta)
    out = _sc_gather_rows(f, ids_flat)
    return out.reshape(B, L, HID)
